# baseline (speedup 1.0000x reference)
import jax
import jax.numpy as jnp
from jax import lax
from jax.experimental import pallas as pl

_B, _N, _D = 512, 32, 64
_BB = 8


def _sswl_kernel(x_idx_ref, A_idx_ref, X_idx_ref,
                 x_emb_ref, ea_emb_ref, tup_emb_ref,
                 W_ti_ref, b_ti_ref, W_conv_ref, b_conv_ref,
                 ln_g_ref, ln_b_ref, Wp1_ref, bp1_ref, Wp2_ref, bp2_ref,
                 out_ref):
    f32 = jnp.float32
    N, D, BB = _N, _D, _BB

    xi = x_idx_ref[...]
    oh_x = (xi[:, :, None]
            == lax.broadcasted_iota(jnp.int32, (BB, N, 32), 2)).astype(f32)
    x = oh_x.reshape(BB * N, 32) @ x_emb_ref[...]
    lin_x = x @ W_ti_ref[...] + b_ti_ref[...]

    ai = A_idx_ref[...]
    oh_a = (ai[:, :, :, None]
            == lax.broadcasted_iota(jnp.int32, (BB, N, N, 16), 3)).astype(f32)
    A = (oh_a.reshape(BB * N * N, 16) @ ea_emb_ref[...]).reshape(BB, N, N, D)

    ti = X_idx_ref[...]
    oh_t = (ti[:, :, :, None]
            == lax.broadcasted_iota(jnp.int32, (BB, N, N, 16), 3)).astype(f32)
    Xt = (oh_t.reshape(BB * N * N, 16) @ tup_emb_ref[...]).reshape(BB, N, N, D)

    xb = x.reshape(BB, N, D)
    lb = lin_x.reshape(BB, N, D)
    X = xb[:, None, :, :] * lb[:, :, None, :] * Xt

    bf16 = jnp.bfloat16
    A16 = A.astype(bf16)
    X16 = X.astype(bf16)
    rows = []
    for b in range(BB):
        A_b = A16[b]
        X_b = X16[b]
        m1 = lax.dot_general(A_b, X_b, (((1,), (0,)), ((2,), (2,))),
                             preferred_element_type=f32)
        m2 = lax.dot_general(X_b, A_b, (((1,), (1,)), ((2,), (2,))),
                             preferred_element_type=f32)
        msum = m1 + m2
        h = lax.dot_general(msum, W_conv_ref[...], (((0,), (0,)), ((), ())),
                            preferred_element_type=f32)
        h = h + b_conv_ref[...]

        mu = jnp.mean(h, axis=-1, keepdims=True)
        var = jnp.mean(jnp.square(h - mu), axis=-1, keepdims=True)
        h = (h - mu) * lax.rsqrt(var + 1e-5) * ln_g_ref[...] + ln_b_ref[...]
        tX = jnp.maximum(h, 0.0)

        rows.append(jnp.sum(X[b] + tX, axis=(0, 1), keepdims=True)
                    .reshape(1, D))
    hg = jnp.concatenate(rows, axis=0) * (1.0 / (N * N))

    hid = jnp.maximum(hg @ Wp1_ref[...] + bp1_ref[...], 0.0)
    out_ref[...] = hid @ Wp2_ref[...] + bp2_ref[...]


def kernel(x_idx, A_idx, X_idx, x_emb, ea_emb, tup_emb, W_ti, b_ti,
           W_conv, b_conv, ln_g, ln_b, Wp1, bp1, Wp2, bp2):
    N, D, BB = _N, _D, _BB
    B = x_idx.shape[0]

    x_idx = x_idx.astype(jnp.int32)
    A_idx = A_idx.astype(jnp.int32)
    X_idx = X_idx.astype(jnp.int32)
    b_ti2 = b_ti.reshape(1, D)
    b_conv2 = b_conv.reshape(1, D)
    ln_g2 = ln_g.reshape(1, D)
    ln_b2 = ln_b.reshape(1, D)
    bp12 = bp1.reshape(1, D)
    bp22 = bp2.reshape(1, 1)

    rep = lambda *dims: pl.BlockSpec(dims, lambda i: (0,) * len(dims))
    out = pl.pallas_call(
        _sswl_kernel,
        grid=(B // BB,),
        in_specs=[
            pl.BlockSpec((BB, N), lambda i: (i, 0)),
            pl.BlockSpec((BB, N, N), lambda i: (i, 0, 0)),
            pl.BlockSpec((BB, N, N), lambda i: (i, 0, 0)),
            rep(32, D), rep(16, D), rep(16, D),
            rep(D, D), rep(1, D), rep(D, D), rep(1, D),
            rep(1, D), rep(1, D),
            rep(D, D), rep(1, D), rep(D, 1), rep(1, 1),
        ],
        out_specs=pl.BlockSpec((BB, 1), lambda i: (i, 0)),
        out_shape=jax.ShapeDtypeStruct((B, 1), jnp.float32),
    )(x_idx, A_idx, X_idx, x_emb, ea_emb, tup_emb,
      W_ti, b_ti2, W_conv, b_conv2, ln_g2, ln_b2, Wp1, bp12, Wp2, bp22)
    return out


# speedup vs baseline: 1.0155x; 1.0155x over previous
import jax
import jax.numpy as jnp
from jax import lax
from jax.experimental import pallas as pl

_B, _N, _D = 512, 32, 64
_BB = 8


def _sswl_kernel(x_idx_ref, A_idx_ref, X_idx_ref,
                 x_emb_ref, ea_emb_ref, tup_emb_ref,
                 W_ti_ref, b_ti_ref, W_bd_ref, b_conv_ref,
                 ln_g_ref, ln_b_ref, Wp1_ref, bp1_ref, Wp2_ref, bp2_ref,
                 out_ref):
    f32 = jnp.float32
    N, D, BB = _N, _D, _BB

    xi = x_idx_ref[...]
    oh_x = (xi[:, :, None]
            == lax.broadcasted_iota(jnp.int32, (BB, N, 32), 2)).astype(f32)
    x = oh_x.reshape(BB * N, 32) @ x_emb_ref[...]
    lin_x = x @ W_ti_ref[...] + b_ti_ref[...]

    ai = A_idx_ref[...]
    oh_a = (ai[:, :, :, None]
            == lax.broadcasted_iota(jnp.int32, (BB, N, N, 16), 3)).astype(f32)
    A = (oh_a.reshape(BB * N * N, 16) @ ea_emb_ref[...]).reshape(BB, N, N, D)

    ti = X_idx_ref[...]
    oh_t = (ti[:, :, :, None]
            == lax.broadcasted_iota(jnp.int32, (BB, N, N, 16), 3)).astype(f32)
    Xt = (oh_t.reshape(BB * N * N, 16) @ tup_emb_ref[...]).reshape(BB, N, N, D)

    xb = x.reshape(BB, N, D)
    lb = lin_x.reshape(BB, N, D)
    X = xb[:, None, :, :] * lb[:, :, None, :] * Xt

    bf16 = jnp.bfloat16
    A16 = A.astype(bf16)
    X16 = X.astype(bf16)
    rows = []
    for g in range(0, BB, 4):
        msums = []
        for b in range(g, g + 4):
            A_b = A16[b]
            X_b = X16[b]
            m1 = lax.dot_general(A_b, X_b, (((1,), (0,)), ((2,), (2,))),
                                 preferred_element_type=f32)
            m2 = lax.dot_general(X_b, A_b, (((1,), (1,)), ((2,), (2,))),
                                 preferred_element_type=f32)
            msums.append(m1 + m2)
        pack = jnp.concatenate(msums, axis=0)
        hp = lax.dot_general(pack, W_bd_ref[...], (((0,), (0,)), ((), ())),
                             preferred_element_type=f32)
        for j, b in enumerate(range(g, g + 4)):
            h = hp[:, :, j * D:(j + 1) * D] + b_conv_ref[...]
            mu = jnp.mean(h, axis=-1, keepdims=True)
            var = jnp.mean(jnp.square(h - mu), axis=-1, keepdims=True)
            h = (h - mu) * lax.rsqrt(var + 1e-5) * ln_g_ref[...] + ln_b_ref[...]
            tX = jnp.maximum(h, 0.0)
            rows.append(jnp.sum(X[b] + tX, axis=(0, 1), keepdims=True)
                        .reshape(1, D))
    hg = jnp.concatenate(rows, axis=0) * (1.0 / (N * N))

    hid = jnp.maximum(hg @ Wp1_ref[...] + bp1_ref[...], 0.0)
    out_ref[...] = hid @ Wp2_ref[...] + bp2_ref[...]


def kernel(x_idx, A_idx, X_idx, x_emb, ea_emb, tup_emb, W_ti, b_ti,
           W_conv, b_conv, ln_g, ln_b, Wp1, bp1, Wp2, bp2):
    N, D, BB = _N, _D, _BB
    B = x_idx.shape[0]

    x_idx = x_idx.astype(jnp.int32)
    A_idx = A_idx.astype(jnp.int32)
    X_idx = X_idx.astype(jnp.int32)
    b_ti2 = b_ti.reshape(1, D)
    b_conv2 = b_conv.reshape(1, D)
    ln_g2 = ln_g.reshape(1, D)
    ln_b2 = ln_b.reshape(1, D)
    bp12 = bp1.reshape(1, D)
    bp22 = bp2.reshape(1, 1)
    Z = jnp.zeros((D, D), jnp.float32)
    W_bd = jnp.concatenate(
        [jnp.concatenate([W_conv if i == j else Z for j in range(4)], axis=1)
         for i in range(4)], axis=0)

    rep = lambda *dims: pl.BlockSpec(dims, lambda i: (0,) * len(dims))
    out = pl.pallas_call(
        _sswl_kernel,
        grid=(B // BB,),
        in_specs=[
            pl.BlockSpec((BB, N), lambda i: (i, 0)),
            pl.BlockSpec((BB, N, N), lambda i: (i, 0, 0)),
            pl.BlockSpec((BB, N, N), lambda i: (i, 0, 0)),
            rep(32, D), rep(16, D), rep(16, D),
            rep(D, D), rep(1, D), rep(4 * D, 4 * D), rep(1, D),
            rep(1, D), rep(1, D),
            rep(D, D), rep(1, D), rep(D, 1), rep(1, 1),
        ],
        out_specs=pl.BlockSpec((BB, 1), lambda i: (i, 0)),
        out_shape=jax.ShapeDtypeStruct((B, 1), jnp.float32),
    )(x_idx, A_idx, X_idx, x_emb, ea_emb, tup_emb,
      W_ti, b_ti2, W_bd, b_conv2, ln_g2, ln_b2, Wp1, bp12, Wp2, bp22)
    return out


# speedup vs baseline: 1.2504x; 1.2314x over previous
import jax
import jax.numpy as jnp
from jax import lax
from jax.experimental import pallas as pl

_B, _N, _D = 512, 32, 64
_BB = 16


def _sswl_kernel(x_idx_ref, A_idx_ref, X_idx_ref,
                 x_emb_ref, ea_emb_ref, tup_emb_ref,
                 W_ti_ref, b_ti_ref, W_bd_ref, b_conv_ref,
                 ln_g_ref, ln_b_ref, Wp1_ref, bp1_ref, Wp2_ref, bp2_ref,
                 out_ref):
    f32 = jnp.float32
    N, D, BB = _N, _D, _BB

    xi = x_idx_ref[...]
    oh_x = (xi[:, :, None]
            == lax.broadcasted_iota(jnp.int32, (BB, N, 32), 2)).astype(f32)
    x = oh_x.reshape(BB * N, 32) @ x_emb_ref[...]
    lin_x = x @ W_ti_ref[...] + b_ti_ref[...]

    ai = A_idx_ref[...]
    oh_a = (ai[:, :, :, None]
            == lax.broadcasted_iota(jnp.int32, (BB, N, N, 16), 3)).astype(f32)
    A = (oh_a.reshape(BB * N * N, 16) @ ea_emb_ref[...]).reshape(BB, N, N, D)

    ti = X_idx_ref[...]
    oh_t = (ti[:, :, :, None]
            == lax.broadcasted_iota(jnp.int32, (BB, N, N, 16), 3)).astype(f32)
    Xt = (oh_t.reshape(BB * N * N, 16) @ tup_emb_ref[...]).reshape(BB, N, N, D)

    xb = x.reshape(BB, N, D)
    lb = lin_x.reshape(BB, N, D)
    X = xb[:, None, :, :] * lb[:, :, None, :] * Xt

    bf16 = jnp.bfloat16
    A16 = A.astype(bf16)
    X16 = X.astype(bf16)
    rows = []
    for g in range(0, BB, 4):
        msums = []
        for b in range(g, g + 4):
            A_b = A16[b]
            X_b = X16[b]
            m1 = lax.dot_general(A_b, X_b, (((1,), (0,)), ((2,), (2,))),
                                 preferred_element_type=f32)
            m2 = lax.dot_general(X_b, A_b, (((1,), (1,)), ((2,), (2,))),
                                 preferred_element_type=f32)
            msums.append(m1 + m2)
        pack = jnp.concatenate(msums, axis=0)
        hp = lax.dot_general(pack, W_bd_ref[...], (((0,), (0,)), ((), ())),
                             preferred_element_type=f32)
        for j, b in enumerate(range(g, g + 4)):
            hc = hp[:, :, j * D:(j + 1) * D] + b_conv_ref[...]
            var = jnp.mean(jnp.square(hc), axis=-1, keepdims=True)
            h = hc * lax.rsqrt(var + 1e-5) * ln_g_ref[...] + ln_b_ref[...]
            tX = jnp.maximum(h, 0.0)
            rows.append(jnp.sum(X[b] + tX, axis=(0, 1), keepdims=True)
                        .reshape(1, D))
    hg = jnp.concatenate(rows, axis=0) * (1.0 / (N * N))

    hid = jnp.maximum(hg @ Wp1_ref[...] + bp1_ref[...], 0.0)
    out_ref[...] = hid @ Wp2_ref[...] + bp2_ref[...]


def kernel(x_idx, A_idx, X_idx, x_emb, ea_emb, tup_emb, W_ti, b_ti,
           W_conv, b_conv, ln_g, ln_b, Wp1, bp1, Wp2, bp2):
    N, D, BB = _N, _D, _BB
    B = x_idx.shape[0]

    x_idx = x_idx.astype(jnp.int32)
    A_idx = A_idx.astype(jnp.int32)
    X_idx = X_idx.astype(jnp.int32)
    b_ti2 = b_ti.reshape(1, D)
    ln_g2 = ln_g.reshape(1, D)
    ln_b2 = ln_b.reshape(1, D)
    bp12 = bp1.reshape(1, D)
    bp22 = bp2.reshape(1, 1)
    Wc = W_conv - jnp.mean(W_conv, axis=1, keepdims=True)
    b_conv2 = (b_conv - jnp.mean(b_conv)).reshape(1, D)
    Z = jnp.zeros((D, D), jnp.float32)
    W_bd = jnp.concatenate(
        [jnp.concatenate([Wc if i == j else Z for j in range(4)], axis=1)
         for i in range(4)], axis=0)

    rep = lambda *dims: pl.BlockSpec(dims, lambda i: (0,) * len(dims))
    out = pl.pallas_call(
        _sswl_kernel,
        grid=(B // BB,),
        in_specs=[
            pl.BlockSpec((BB, N), lambda i: (i, 0)),
            pl.BlockSpec((BB, N, N), lambda i: (i, 0, 0)),
            pl.BlockSpec((BB, N, N), lambda i: (i, 0, 0)),
            rep(32, D), rep(16, D), rep(16, D),
            rep(D, D), rep(1, D), rep(4 * D, 4 * D), rep(1, D),
            rep(1, D), rep(1, D),
            rep(D, D), rep(1, D), rep(D, 1), rep(1, 1),
        ],
        out_specs=pl.BlockSpec((BB, 1), lambda i: (i, 0)),
        out_shape=jax.ShapeDtypeStruct((B, 1), jnp.float32),
    )(x_idx, A_idx, X_idx, x_emb, ea_emb, tup_emb,
      W_ti, b_ti2, W_bd, b_conv2, ln_g2, ln_b2, Wp1, bp12, Wp2, bp22)
    return out
